# baseline (speedup 1.0000x reference)
import functools

import jax
import jax.numpy as jnp
from jax.experimental import pallas as pl
from jax.experimental.pallas import tpu as pltpu

_NH = 8
_MB = 4096
_TOPK = 16


def _scores(Q, keys_blk, ipw_ref, ipb_ref, E, hd):
    wk = ipw_ref[E:2 * E, :]
    bk = ipb_ref[:, E:2 * E]
    Kb = jax.lax.dot_general(keys_blk, wk, (((1,), (1,)), ((), ())),
                             preferred_element_type=jnp.float32) + bk
    inv = jnp.sqrt(jnp.float32(hd))
    rows = []
    for hh in range(_NH):
        Qh = Q[:, hh * hd:(hh + 1) * hd]
        Kh = Kb[:, hh * hd:(hh + 1) * hd]
        rows.append(jax.lax.dot_general(
            Qh, Kh, (((1,), (1,)), ((), ())),
            preferred_element_type=jnp.float32) / inv)
    return rows


def _attn_stats_kernel(nblk, M, BQ, E,
                       q_ref, w1_ref, b1_ref, w2_ref, b2_ref,
                       ipw_ref, ipb_ref, opw_ref, opb_ref,
                       keys_ref, vals_ref,
                       att_ref, q_out_ref, m_out_ref, l_out_ref,
                       q_s, m_s, l_s, u_s):
    hd = E // _NH
    i = pl.program_id(0)

    @pl.when(i == 0)
    def _prologue():
        q = q_ref[:]
        h = jnp.maximum(
            jax.lax.dot_general(q, w1_ref[:], (((1,), (1,)), ((), ())),
                                preferred_element_type=jnp.float32)
            + b1_ref[:], 0.0)
        qk = jax.lax.dot_general(h, w2_ref[:], (((1,), (1,)), ((), ())),
                                 preferred_element_type=jnp.float32) + b2_ref[:]
        wq = ipw_ref[0:E, :]
        bq = ipb_ref[:, 0:E]
        q_s[:] = jax.lax.dot_general(qk, wq, (((1,), (1,)), ((), ())),
                                     preferred_element_type=jnp.float32) + bq
        m_s[:] = jnp.full_like(m_s[:], -1e30)
        l_s[:] = jnp.zeros_like(l_s[:])
        u_s[:] = jnp.zeros_like(u_s[:])

    S = jnp.concatenate(
        _scores(q_s[:], keys_ref[:], ipw_ref, ipb_ref, E, hd), axis=0)
    col = jax.lax.broadcasted_iota(jnp.int32, (1, _MB), 1) + i * _MB
    S = jnp.where(col < M, S, -1e30)
    m_new = jnp.maximum(m_s[:], jnp.max(S, axis=1, keepdims=True))
    alpha = jnp.exp(m_s[:] - m_new)
    P = jnp.exp(S - m_new)
    l_s[:] = l_s[:] * alpha + jnp.sum(P, axis=1, keepdims=True)
    row = jax.lax.broadcasted_iota(jnp.int32, (_MB, 1), 0) + i * _MB
    vals = jnp.where(row < M, vals_ref[:], 0.0)
    u_s[:] = u_s[:] * alpha + jnp.dot(P, vals,
                                      preferred_element_type=jnp.float32)
    m_s[:] = m_new

    @pl.when(i == nblk - 1)
    def _epilogue():
        r = 1.0 / l_s[:]
        cols = []
        for hh in range(_NH):
            Ah = u_s[hh * BQ:(hh + 1) * BQ, :] * r[hh * BQ:(hh + 1) * BQ, :]
            wvh = ipw_ref[2 * E + hh * hd:2 * E + (hh + 1) * hd, :]
            bvh = ipb_ref[:, 2 * E + hh * hd:2 * E + (hh + 1) * hd]
            cols.append(jax.lax.dot_general(
                Ah, wvh, (((1,), (1,)), ((), ())),
                preferred_element_type=jnp.float32) + bvh)
        out = jnp.concatenate(cols, axis=1)
        att_ref[:] = jax.lax.dot_general(
            out, opw_ref[:], (((1,), (1,)), ((), ())),
            preferred_element_type=jnp.float32) + opb_ref[:]
        q_out_ref[:] = q_s[:]
        m_out_ref[:] = m_s[:]
        l_out_ref[:] = l_s[:]


def _topk_body(nblk, M, BQ, E, mb, q_ref, m_ref, l_ref, ipw_ref, ipb_ref,
               keys_ref, w_ref, i_ref):
    del nblk
    hd = E // _NH
    i = pl.program_id(0)
    rows = _scores(q_ref[:], keys_ref[:], ipw_ref, ipb_ref, E, hd)
    aw = None
    for hh in range(_NH):
        x = (jnp.exp(rows[hh] - m_ref[hh * BQ:(hh + 1) * BQ, :])
             / l_ref[hh * BQ:(hh + 1) * BQ, :])
        aw = x if aw is None else aw + x
    aw = aw * (1.0 / _NH)
    coli = jax.lax.broadcasted_iota(jnp.int32, (BQ, mb), 1)
    base = i * mb
    aw = jnp.where(coli + base < M, aw, -1.0)
    iotaf = coli.astype(jnp.float32)

    @pl.when(i == 0)
    def _init():
        w_ref[:] = jnp.full((BQ, _TOPK), -3.0, jnp.float32)
        i_ref[:] = jnp.zeros((BQ, _TOPK), jnp.int32)

    def _cond(state):
        _, w, _, m = state
        return jnp.any(m > w[:, _TOPK - 1:_TOPK])

    def _body(state):
        v, w, idxf, m = state
        ar = jnp.min(jnp.where(v == m, iotaf, 1e9), axis=1, keepdims=True)
        gidx = ar + base
        keep = w >= m
        w_shift = jnp.concatenate([w[:, :1], w[:, :_TOPK - 1]], axis=1)
        colpos = jax.lax.broadcasted_iota(jnp.int32, (BQ, _TOPK), 1)
        keep_prev = (w_shift >= m) | (colpos == 0)
        i_shift = jnp.concatenate([idxf[:, :1], idxf[:, :_TOPK - 1]], axis=1)
        w_new = jnp.where(keep, w, jnp.where(keep_prev, m, w_shift))
        i_new = jnp.where(keep, idxf, jnp.where(keep_prev, gidx, i_shift))
        v_new = jnp.where(iotaf == ar, -1.0, v)
        m_new = jnp.max(v_new, axis=1, keepdims=True)
        return v_new, w_new, i_new, m_new

    state = (aw, w_ref[:], i_ref[:].astype(jnp.float32),
             jnp.max(aw, axis=1, keepdims=True))
    _, w_fin, i_fin, _ = jax.lax.while_loop(_cond, _body, state)
    w_ref[:] = w_fin
    i_ref[:] = i_fin.astype(jnp.int32)


def _run(query, W1, b1, W2, b2, memory_keys, memory_values,
         in_proj_w, in_proj_b, out_proj_w, out_proj_b):
    BQ, E = query.shape
    M = memory_keys.shape[0]
    R = _NH * BQ
    nblk = (M + _MB - 1) // _MB
    f32 = jnp.float32

    b1_2 = b1.reshape(1, -1)
    b2_2 = b2.reshape(1, -1)
    ipb_2 = in_proj_b.reshape(1, -1)
    opb_2 = out_proj_b.reshape(1, -1)

    const = lambda shape: pl.BlockSpec(shape, lambda i: (0, 0))
    stream = pl.BlockSpec((_MB, E), lambda i: (i, 0))

    att, q_enc, m, l = pl.pallas_call(
        functools.partial(_attn_stats_kernel, nblk, M, BQ, E),
        grid=(nblk,),
        in_specs=[
            const((BQ, E)), const(W1.shape), const((1, b1.shape[0])),
            const(W2.shape), const((1, b2.shape[0])),
            const(in_proj_w.shape), const((1, in_proj_b.shape[0])),
            const(out_proj_w.shape), const((1, out_proj_b.shape[0])),
            stream, stream,
        ],
        out_specs=[const((BQ, E)), const((BQ, E)), const((R, 1)),
                   const((R, 1))],
        out_shape=[
            jax.ShapeDtypeStruct((BQ, E), f32),
            jax.ShapeDtypeStruct((BQ, E), f32),
            jax.ShapeDtypeStruct((R, 1), f32),
            jax.ShapeDtypeStruct((R, 1), f32),
        ],
        scratch_shapes=[
            pltpu.VMEM((BQ, E), f32), pltpu.VMEM((R, 1), f32),
            pltpu.VMEM((R, 1), f32), pltpu.VMEM((R, E), f32),
        ],
    )(query, W1, b1_2, W2, b2_2, in_proj_w, ipb_2, out_proj_w, opb_2,
      memory_keys, memory_values)

    mbb = 8192
    nblkb = (M + mbb - 1) // mbb
    topw, topi = pl.pallas_call(
        functools.partial(_topk_body, nblkb, M, BQ, E, mbb),
        grid=(nblkb,),
        in_specs=[const((BQ, E)), const((R, 1)), const((R, 1)),
                  const(in_proj_w.shape), const((1, in_proj_b.shape[0])),
                  pl.BlockSpec((mbb, E), lambda i: (i, 0))],
        out_specs=[const((BQ, _TOPK)), const((BQ, _TOPK))],
        out_shape=[
            jax.ShapeDtypeStruct((BQ, _TOPK), f32),
            jax.ShapeDtypeStruct((BQ, _TOPK), jnp.int32),
        ],
    )(q_enc, m, l, in_proj_w, ipb_2, memory_keys)

    return att.reshape(1, BQ, E), topw, topi


def kernel(query, k, W1, b1, W2, b2, memory_keys, memory_values,
           in_proj_w, in_proj_b, out_proj_w, out_proj_b):
    del k
    return _run(query, W1, b1, W2, b2, memory_keys, memory_values,
                in_proj_w, in_proj_b, out_proj_w, out_proj_b)


# speedup vs baseline: 1.0333x; 1.0333x over previous
import functools

import jax
import jax.numpy as jnp
from jax.experimental import pallas as pl
from jax.experimental.pallas import tpu as pltpu

_NH = 8
_MB = 4096
_TOPK = 16


def _scores(Q, keys_blk, ipw_ref, ipb_ref, E, hd):
    wk = ipw_ref[E:2 * E, :]
    bk = ipb_ref[:, E:2 * E]
    Kb = jax.lax.dot_general(keys_blk, wk, (((1,), (1,)), ((), ())),
                             preferred_element_type=jnp.float32) + bk
    inv = jnp.sqrt(jnp.float32(hd))
    rows = []
    for hh in range(_NH):
        Qh = Q[:, hh * hd:(hh + 1) * hd]
        Kh = Kb[:, hh * hd:(hh + 1) * hd]
        rows.append(jax.lax.dot_general(
            Qh, Kh, (((1,), (1,)), ((), ())),
            preferred_element_type=jnp.float32) / inv)
    return rows


def _attn_stats_kernel(nblk, M, BQ, E,
                       q_ref, w1_ref, b1_ref, w2_ref, b2_ref,
                       ipw_ref, ipb_ref, opw_ref, opb_ref,
                       keys_ref, vals_ref,
                       att_ref, q_out_ref, m_out_ref, l_out_ref,
                       q_s, m_s, l_s, u_s):
    hd = E // _NH
    i = pl.program_id(0)

    @pl.when(i == 0)
    def _prologue():
        q = q_ref[:]
        h = jnp.maximum(
            jax.lax.dot_general(q, w1_ref[:], (((1,), (1,)), ((), ())),
                                preferred_element_type=jnp.float32)
            + b1_ref[:], 0.0)
        qk = jax.lax.dot_general(h, w2_ref[:], (((1,), (1,)), ((), ())),
                                 preferred_element_type=jnp.float32) + b2_ref[:]
        wq = ipw_ref[0:E, :]
        bq = ipb_ref[:, 0:E]
        q_s[:] = jax.lax.dot_general(qk, wq, (((1,), (1,)), ((), ())),
                                     preferred_element_type=jnp.float32) + bq
        m_s[:] = jnp.full_like(m_s[:], -1e30)
        l_s[:] = jnp.zeros_like(l_s[:])
        u_s[:] = jnp.zeros_like(u_s[:])

    S = jnp.concatenate(
        _scores(q_s[:], keys_ref[:], ipw_ref, ipb_ref, E, hd), axis=0)
    col = jax.lax.broadcasted_iota(jnp.int32, (1, _MB), 1) + i * _MB
    S = jnp.where(col < M, S, -1e30)
    m_new = jnp.maximum(m_s[:], jnp.max(S, axis=1, keepdims=True))
    alpha = jnp.exp(m_s[:] - m_new)
    P = jnp.exp(S - m_new)
    l_s[:] = l_s[:] * alpha + jnp.sum(P, axis=1, keepdims=True)
    row = jax.lax.broadcasted_iota(jnp.int32, (_MB, 1), 0) + i * _MB
    vals = jnp.where(row < M, vals_ref[:], 0.0)
    u_s[:] = u_s[:] * alpha + jnp.dot(P, vals,
                                      preferred_element_type=jnp.float32)
    m_s[:] = m_new

    @pl.when(i == nblk - 1)
    def _epilogue():
        r = 1.0 / l_s[:]
        cols = []
        for hh in range(_NH):
            Ah = u_s[hh * BQ:(hh + 1) * BQ, :] * r[hh * BQ:(hh + 1) * BQ, :]
            wvh = ipw_ref[2 * E + hh * hd:2 * E + (hh + 1) * hd, :]
            bvh = ipb_ref[:, 2 * E + hh * hd:2 * E + (hh + 1) * hd]
            cols.append(jax.lax.dot_general(
                Ah, wvh, (((1,), (1,)), ((), ())),
                preferred_element_type=jnp.float32) + bvh)
        out = jnp.concatenate(cols, axis=1)
        att_ref[:] = jax.lax.dot_general(
            out, opw_ref[:], (((1,), (1,)), ((), ())),
            preferred_element_type=jnp.float32) + opb_ref[:]
        q_out_ref[:] = q_s[:]
        m_out_ref[:] = m_s[:]
        l_out_ref[:] = l_s[:]


def _topk_body(nblk, M, BQ, E, mb, q_ref, m_ref, l_ref, ipw_ref, ipb_ref,
               keys_ref, w_ref, i_ref):
    del nblk
    hd = E // _NH
    i = pl.program_id(0)
    rows = _scores(q_ref[:], keys_ref[:], ipw_ref, ipb_ref, E, hd)
    aw = None
    for hh in range(_NH):
        x = (jnp.exp(rows[hh] - m_ref[hh * BQ:(hh + 1) * BQ, :])
             / l_ref[hh * BQ:(hh + 1) * BQ, :])
        aw = x if aw is None else aw + x
    aw = aw * (1.0 / _NH)
    coli = jax.lax.broadcasted_iota(jnp.int32, (BQ, mb), 1)
    base = i * mb
    aw = jnp.where(coli + base < M, aw, -1.0)
    iotaf = coli.astype(jnp.float32)

    @pl.when(i == 0)
    def _init():
        w_ref[:] = jnp.full((BQ, _TOPK), -3.0, jnp.float32)
        i_ref[:] = jnp.zeros((BQ, _TOPK), jnp.int32)

    def _cond(state):
        _, w, _, m = state
        return jnp.any(m > w[:, _TOPK - 1:_TOPK])

    def _body(state):
        v, w, idxf, m = state
        ar = jnp.min(jnp.where(v == m, iotaf, 1e9), axis=1, keepdims=True)
        gidx = ar + base
        keep = w >= m
        w_shift = jnp.concatenate([w[:, :1], w[:, :_TOPK - 1]], axis=1)
        colpos = jax.lax.broadcasted_iota(jnp.int32, (BQ, _TOPK), 1)
        keep_prev = (w_shift >= m) | (colpos == 0)
        i_shift = jnp.concatenate([idxf[:, :1], idxf[:, :_TOPK - 1]], axis=1)
        w_new = jnp.where(keep, w, jnp.where(keep_prev, m, w_shift))
        i_new = jnp.where(keep, idxf, jnp.where(keep_prev, gidx, i_shift))
        v_new = jnp.where(iotaf == ar, -1.0, v)
        m_new = jnp.max(v_new, axis=1, keepdims=True)
        return v_new, w_new, i_new, m_new

    state = (aw, w_ref[:], i_ref[:].astype(jnp.float32),
             jnp.max(aw, axis=1, keepdims=True))
    _, w_fin, i_fin, _ = jax.lax.while_loop(_cond, _body, state)
    w_ref[:] = w_fin
    i_ref[:] = i_fin.astype(jnp.int32)


def _run(query, W1, b1, W2, b2, memory_keys, memory_values,
         in_proj_w, in_proj_b, out_proj_w, out_proj_b):
    BQ, E = query.shape
    M = memory_keys.shape[0]
    R = _NH * BQ
    nblk = (M + _MB - 1) // _MB
    f32 = jnp.float32

    b1_2 = b1.reshape(1, -1)
    b2_2 = b2.reshape(1, -1)
    ipb_2 = in_proj_b.reshape(1, -1)
    opb_2 = out_proj_b.reshape(1, -1)

    const = lambda shape: pl.BlockSpec(shape, lambda i: (0, 0))
    stream = pl.BlockSpec((_MB, E), lambda i: (i, 0))

    att, q_enc, m, l = pl.pallas_call(
        functools.partial(_attn_stats_kernel, nblk, M, BQ, E),
        grid=(nblk,),
        in_specs=[
            const((BQ, E)), const(W1.shape), const((1, b1.shape[0])),
            const(W2.shape), const((1, b2.shape[0])),
            const(in_proj_w.shape), const((1, in_proj_b.shape[0])),
            const(out_proj_w.shape), const((1, out_proj_b.shape[0])),
            stream, stream,
        ],
        out_specs=[const((BQ, E)), const((BQ, E)), const((R, 1)),
                   const((R, 1))],
        out_shape=[
            jax.ShapeDtypeStruct((BQ, E), f32),
            jax.ShapeDtypeStruct((BQ, E), f32),
            jax.ShapeDtypeStruct((R, 1), f32),
            jax.ShapeDtypeStruct((R, 1), f32),
        ],
        scratch_shapes=[
            pltpu.VMEM((BQ, E), f32), pltpu.VMEM((R, 1), f32),
            pltpu.VMEM((R, 1), f32), pltpu.VMEM((R, E), f32),
        ],
    )(query, W1, b1_2, W2, b2_2, in_proj_w, ipb_2, out_proj_w, opb_2,
      memory_keys, memory_values)

    mbb = 4096
    nblkb = (M + mbb - 1) // mbb
    topw, topi = pl.pallas_call(
        functools.partial(_topk_body, nblkb, M, BQ, E, mbb),
        grid=(nblkb,),
        in_specs=[const((BQ, E)), const((R, 1)), const((R, 1)),
                  const(in_proj_w.shape), const((1, in_proj_b.shape[0])),
                  pl.BlockSpec((mbb, E), lambda i: (i, 0))],
        out_specs=[const((BQ, _TOPK)), const((BQ, _TOPK))],
        out_shape=[
            jax.ShapeDtypeStruct((BQ, _TOPK), f32),
            jax.ShapeDtypeStruct((BQ, _TOPK), jnp.int32),
        ],
    )(q_enc, m, l, in_proj_w, ipb_2, memory_keys)

    return att.reshape(1, BQ, E), topw, topi


def kernel(query, k, W1, b1, W2, b2, memory_keys, memory_values,
           in_proj_w, in_proj_b, out_proj_w, out_proj_b):
    del k
    return _run(query, W1, b1, W2, b2, memory_keys, memory_values,
                in_proj_w, in_proj_b, out_proj_w, out_proj_b)
